# baseline (speedup 1.0000x reference)
import functools

import jax
import jax.numpy as jnp
from jax import lax
from jax.experimental import pallas as pl
from jax.experimental.pallas import tpu as pltpu
from jax.experimental.pallas import tpu_sc as plsc

BATCH = 16384
NF = 26
VOCAB = 100000
D = 32
ROWS = BATCH * NF
NC = 2
NS = 16
NW = NC * NS
RPW = ROWS // NW
CH = 1664
NCH = RPW // CH
LANES = 16
BW = 128
NBT = BATCH // BW
BT_PER_W = NBT // NW
BLK = NF * BW
INV13 = -991146299

_mesh = plsc.VectorSubcoreMesh(core_axis_name="c", subcore_axis_name="s")


@functools.partial(
    pl.kernel,
    mesh=_mesh,
    out_type=jax.ShapeDtypeStruct((ROWS,), jnp.int32),
    compiler_params=pltpu.CompilerParams(
        use_tc_tiling_on_sc=True, needs_layout_passes=False
    ),
    scratch_types=[
        pltpu.VMEM((32, BW), jnp.int32),
        pltpu.VMEM((BLK,), jnp.int32),
    ],
)
def _idx_prep(idxt_hbm, tail_hbm, gidx_hbm, slab_v, buf_v):
    wid = lax.axis_index("s") * NC + lax.axis_index("c")
    lane = lax.iota(jnp.int32, LANES)
    fpats = tuple(lax.rem(k * LANES + lane, NF) for k in range(13))

    for bt_i in range(BT_PER_W):
        bt = wid * BT_PER_W + bt_i
        pltpu.sync_copy(
            idxt_hbm.at[pl.ds(0, 24), pl.ds(bt * BW, BW)],
            slab_v.at[pl.ds(0, 24)],
        )
        pltpu.sync_copy(
            tail_hbm.at[:, pl.ds(bt * BW, BW)], slab_v.at[pl.ds(24, 8)]
        )

        def body(g, carry):
            for kk in range(13):
                k = g * 13 + kk
                f = carry[kk]
                p = k * LANES + lane
                j = lax.shift_right_logical(p - f, 1) * jnp.int32(INV13)
                v = plsc.load_gather(slab_v, [f, j])
                buf_v[pl.ds(k * LANES, LANES)] = v + f * VOCAB
            return carry

        lax.fori_loop(0, 16, body, fpats)
        pltpu.sync_copy(buf_v, gidx_hbm.at[pl.ds(bt * BLK, BLK)])


NVT = VOCAB // BW
NGRP = NF * (NVT + 1)
GPW_LO = NGRP // NW
GREM = NGRP - GPW_LO * NW


@functools.partial(
    pl.kernel,
    mesh=_mesh,
    out_type=jax.ShapeDtypeStruct((NF * VOCAB * D,), jnp.float32),
    compiler_params=pltpu.CompilerParams(
        use_tc_tiling_on_sc=True, needs_layout_passes=False
    ),
    scratch_types=[
        pltpu.VMEM((D, BW), jnp.float32),
        pltpu.VMEM((BW * D,), jnp.float32),
    ],
)
def _table_fmt(tabt_hbm, tabtail_hbm, flat_hbm, slab_v, obuf_v):
    wid = lax.axis_index("s") * NC + lax.axis_index("c")
    lane = lax.iota(jnp.int32, LANES)
    dlo = lane
    dhi = lane + LANES
    lo = wid * GPW_LO + lax.min(wid, GREM)
    hi = lo + GPW_LO + jnp.where(wid < GREM, 1, 0)

    def col(v, c):
        obuf_v[pl.ds(v * D, LANES)] = plsc.load_gather(
            slab_v, [dlo, jnp.full((LANES,), v, jnp.int32)]
        )
        obuf_v[pl.ds(v * D + LANES, LANES)] = plsc.load_gather(
            slab_v, [dhi, jnp.full((LANES,), v, jnp.int32)]
        )
        return c

    VTAIL = VOCAB - NVT * BW

    def group(gg, carry):
        f = gg // (NVT + 1)
        vt = gg % (NVT + 1)
        is_tail = vt == NVT

        @pl.when(jnp.logical_not(is_tail))
        def _():
            pltpu.sync_copy(tabt_hbm.at[f, :, pl.ds(vt * BW, BW)], slab_v)
            lax.fori_loop(0, BW, col, 0)
            pltpu.sync_copy(
                obuf_v,
                flat_hbm.at[pl.ds((f * VOCAB + vt * BW) * D, BW * D)],
            )

        @pl.when(is_tail)
        def _():
            pltpu.sync_copy(tabtail_hbm.at[f], slab_v)
            lax.fori_loop(0, VTAIL, col, 0)
            pltpu.sync_copy(
                obuf_v.at[pl.ds(0, VTAIL * D)],
                flat_hbm.at[pl.ds((f * VOCAB + NVT * BW) * D, VTAIL * D)],
            )

        return carry

    lax.fori_loop(lo, hi, group, 0)


@functools.partial(
    pl.kernel,
    mesh=_mesh,
    out_type=jax.ShapeDtypeStruct((ROWS, D), jnp.float32),
    compiler_params=pltpu.CompilerParams(use_tc_tiling_on_sc=False),
    scratch_types=[
        pltpu.VMEM((RPW,), jnp.int32),
        pltpu.VMEM((CH, D), jnp.float32),
        pltpu.SemaphoreType.DMA,
    ],
)
def _emb_gather(idx_hbm, table_hbm, out_hbm, idx_v, rows_v, sem):
    wid = lax.axis_index("s") * NC + lax.axis_index("c")
    base = wid * RPW
    pltpu.sync_copy(idx_hbm.at[pl.ds(base, RPW)], idx_v)

    for c in range(NCH):
        pltpu.async_copy(
            table_hbm.at[idx_v.at[pl.ds(c * CH, CH)]], rows_v, sem
        ).wait()
        pltpu.sync_copy(rows_v, out_hbm.at[pl.ds(base + c * CH, CH)])


def kernel(indices, tables):
    idxt = jnp.swapaxes(indices.astype(jnp.int32), 0, 1)
    tail = jnp.pad(idxt[24:26], ((0, 6), (0, 0)))
    gidx = _idx_prep(idxt, tail)
    tabt = jnp.transpose(tables, (0, 2, 1))
    tabtail = jnp.pad(tabt[:, :, NVT * BW:], ((0, 0), (0, 0), (0, BW - VOCAB % BW)))
    flat_tables = _table_fmt(tabt, tabtail).reshape(NF * VOCAB, D)
    out = _emb_gather(gidx, flat_tables)
    return out.reshape(BATCH, NF * D)


# speedup vs baseline: 2.1328x; 2.1328x over previous
import functools

import jax
import jax.numpy as jnp
from jax import lax
from jax.experimental import pallas as pl
from jax.experimental.pallas import tpu as pltpu
from jax.experimental.pallas import tpu_sc as plsc

BATCH = 16384
NF = 26
VOCAB = 100000
D = 32
ROWS = BATCH * NF
NC = 2
NS = 16
NW = NC * NS
RPW = ROWS // NW
CH = 1664
NCH = RPW // CH
LANES = 16
BW = 128
NBT = BATCH // BW
BT_PER_W = NBT // NW
BLK = NF * BW
INV13 = -991146299

_mesh = plsc.VectorSubcoreMesh(core_axis_name="c", subcore_axis_name="s")


@functools.partial(
    pl.kernel,
    mesh=_mesh,
    out_type=jax.ShapeDtypeStruct((ROWS,), jnp.int32),
    compiler_params=pltpu.CompilerParams(
        use_tc_tiling_on_sc=True, needs_layout_passes=False
    ),
    scratch_types=[
        pltpu.VMEM((32, BW), jnp.int32),
        pltpu.VMEM((BLK,), jnp.int32),
    ],
)
def _idx_prep(idxt_hbm, tail_hbm, gidx_hbm, slab_v, buf_v):
    wid = lax.axis_index("s") * NC + lax.axis_index("c")
    lane = lax.iota(jnp.int32, LANES)
    fpats = tuple(lax.rem(k * LANES + lane, NF) for k in range(13))

    for bt_i in range(BT_PER_W):
        bt = wid * BT_PER_W + bt_i
        pltpu.sync_copy(
            idxt_hbm.at[pl.ds(0, 24), pl.ds(bt * BW, BW)],
            slab_v.at[pl.ds(0, 24)],
        )
        pltpu.sync_copy(
            tail_hbm.at[:, pl.ds(bt * BW, BW)], slab_v.at[pl.ds(24, 8)]
        )

        def body(g, carry):
            for kk in range(13):
                k = g * 13 + kk
                f = carry[kk]
                p = k * LANES + lane
                j = lax.shift_right_logical(p - f, 1) * jnp.int32(INV13)
                v = plsc.load_gather(slab_v, [f, j])
                buf_v[pl.ds(k * LANES, LANES)] = v + f * VOCAB
            return carry

        lax.fori_loop(0, 16, body, fpats)
        pltpu.sync_copy(buf_v, gidx_hbm.at[pl.ds(bt * BLK, BLK)])


@functools.partial(
    pl.kernel,
    mesh=_mesh,
    out_type=jax.ShapeDtypeStruct((ROWS, D), jnp.float32),
    compiler_params=pltpu.CompilerParams(use_tc_tiling_on_sc=False),
    scratch_types=[
        pltpu.VMEM((RPW,), jnp.int32),
        pltpu.VMEM((CH, D), jnp.float32),
        pltpu.SemaphoreType.DMA,
    ],
)
def _emb_gather(idx_hbm, table_hbm, out_hbm, idx_v, rows_v, sem):
    wid = lax.axis_index("s") * NC + lax.axis_index("c")
    base = wid * RPW
    pltpu.sync_copy(idx_hbm.at[pl.ds(base, RPW)], idx_v)

    for c in range(NCH):
        pltpu.async_copy(
            table_hbm.at[idx_v.at[pl.ds(c * CH, CH)]], rows_v, sem
        ).wait()
        pltpu.sync_copy(rows_v, out_hbm.at[pl.ds(base + c * CH, CH)])


def kernel(indices, tables):
    idxt = jnp.swapaxes(indices.astype(jnp.int32), 0, 1)
    tail = jnp.pad(idxt[24:26], ((0, 6), (0, 0)))
    gidx = _idx_prep(idxt, tail)
    flat_tables = tables.reshape(NF * VOCAB, D)
    out = _emb_gather(gidx, flat_tables)
    return out.reshape(BATCH, NF * D)
